# Initial kernel scaffold; baseline (speedup 1.0000x reference)
import jax
import jax.numpy as jnp
from jax.experimental import pallas as pl


def kernel(x, edge_index, edge_weight, W):
    raise NotImplementedError("write your pallas kernel here")


# speedup vs baseline: 3.6753x; 3.6753x over previous
import functools

import jax
import jax.numpy as jnp
from jax import lax
from jax.experimental import pallas as pl
from jax.experimental.pallas import tpu as pltpu
from jax.experimental.pallas import tpu_sc as plsc

N = 10000
D = 128
E = 320000
NUM_CORES = 2
NUM_SUBCORES = 16
NUM_TILES = NUM_CORES * NUM_SUBCORES
CHUNK = 128
EDGES_PER_TILE = 10112
E_PAD = EDGES_PER_TILE * NUM_TILES
CHUNKS_PER_TILE = EDGES_PER_TILE // CHUNK
ROWS_MAIN = 624
ROWS_LAST = N - ROWS_MAIN * (NUM_SUBCORES - 1)


def _sc_aggregate(x, src, dst, w):
  mesh = plsc.VectorSubcoreMesh(core_axis_name="c", subcore_axis_name="s")

  @functools.partial(
      pl.kernel,
      mesh=mesh,
      out_type=(
          jax.ShapeDtypeStruct((N, D), jnp.float32),
          jax.ShapeDtypeStruct((N, D), jnp.float32),
      ),
      scratch_types=[
          pltpu.VMEM((CHUNK,), jnp.int32),
          pltpu.VMEM((CHUNK,), jnp.int32),
          pltpu.VMEM((CHUNK,), jnp.float32),
          pltpu.VMEM((CHUNK, D), jnp.float32),
          pltpu.VMEM_SHARED((N, D), jnp.float32),
          pltpu.SemaphoreType.DMA,
      ],
  )
  def k(x_hbm, src_hbm, dst_hbm, w_hbm, out0, out1,
        idx_v, dstv, wv, rows, acc, sem):
    c = lax.axis_index("c")
    s = lax.axis_index("s")
    row_base = s * ROWS_MAIN

    def zrow(i, _):
      for j in range(D // 16):
        rows[i, pl.ds(j * 16, 16)] = jnp.zeros((16,), jnp.float32)
      return 0
    lax.fori_loop(0, CHUNK, zrow, 0)

    @pl.when(s < NUM_SUBCORES - 1)
    def _():
      for kk in range(4):
        pltpu.sync_copy(rows, acc.at[pl.ds(row_base + kk * CHUNK, CHUNK)])
      pltpu.sync_copy(rows.at[pl.ds(0, ROWS_MAIN - 4 * CHUNK)],
                      acc.at[pl.ds(row_base + 4 * CHUNK,
                                   ROWS_MAIN - 4 * CHUNK)])

    @pl.when(s == NUM_SUBCORES - 1)
    def _():
      for kk in range(ROWS_LAST // CHUNK):
        pltpu.sync_copy(rows, acc.at[pl.ds(row_base + kk * CHUNK, CHUNK)])
    plsc.subcore_barrier()

    edge_base = (c * NUM_SUBCORES + s) * EDGES_PER_TILE

    def chunk_body(g, _):
      base = edge_base + g * CHUNK
      pltpu.sync_copy(src_hbm.at[pl.ds(base, CHUNK)], idx_v)
      pltpu.sync_copy(dst_hbm.at[pl.ds(base, CHUNK)], dstv)
      pltpu.sync_copy(w_hbm.at[pl.ds(base, CHUNK)], wv)
      pltpu.async_copy(x_hbm.at[idx_v], rows, sem).wait()

      def group(j, _):
        w16 = wv[pl.ds(j * 16, 16)]
        for r in range(16):
          wb = lax.gather(
              w16, jnp.full((16, 1), r, jnp.int32),
              lax.GatherDimensionNumbers(offset_dims=(),
                                         collapsed_slice_dims=(0,),
                                         start_index_map=(0,)),
              (1,), mode=lax.GatherScatterMode.PROMISE_IN_BOUNDS)
          row = j * 16 + r
          for cc in range(D // 16):
            sl = pl.ds(cc * 16, 16)
            rows[row, sl] = rows[row, sl] * wb
        return 0
      lax.fori_loop(0, CHUNK // 16, group, 0)

      pltpu.sync_copy(rows, acc.at[dstv], add=True)
      return 0
    lax.fori_loop(0, CHUNKS_PER_TILE, chunk_body, 0)

    plsc.subcore_barrier()

    def epilogue(out_ref):
      @pl.when(s < NUM_SUBCORES - 1)
      def _():
        pltpu.sync_copy(acc.at[pl.ds(row_base, ROWS_MAIN)],
                        out_ref.at[pl.ds(row_base, ROWS_MAIN)])

      @pl.when(s == NUM_SUBCORES - 1)
      def _():
        pltpu.sync_copy(acc.at[pl.ds(row_base, ROWS_LAST)],
                        out_ref.at[pl.ds(row_base, ROWS_LAST)])

    @pl.when(c == 0)
    def _():
      epilogue(out0)

    @pl.when(c == 1)
    def _():
      epilogue(out1)

  return k(x, src, dst, w)


def _tc_matmul(agg0, agg1, w):
  blk = 1000

  def body(a0_ref, a1_ref, w_ref, o_ref):
    o_ref[...] = jnp.dot(a0_ref[...] + a1_ref[...], w_ref[...],
                         preferred_element_type=jnp.float32)

  return pl.pallas_call(
      body,
      grid=(N // blk,),
      in_specs=[
          pl.BlockSpec((blk, D), lambda i: (i, 0)),
          pl.BlockSpec((blk, D), lambda i: (i, 0)),
          pl.BlockSpec((D, D), lambda i: (0, 0)),
      ],
      out_specs=pl.BlockSpec((blk, D), lambda i: (i, 0)),
      out_shape=jax.ShapeDtypeStruct((N, D), jnp.float32),
  )(agg0, agg1, w)


def kernel(x, edge_index, edge_weight, W):
  src = edge_index[1]
  dst = edge_index[0]
  pad = E_PAD - E
  src = jnp.concatenate([src, jnp.zeros((pad,), jnp.int32)])
  dst = jnp.concatenate([dst, jnp.zeros((pad,), jnp.int32)])
  w = jnp.concatenate([edge_weight, jnp.zeros((pad,), jnp.float32)])
  agg0, agg1 = _sc_aggregate(x, src, dst, w)
  return _tc_matmul(agg0, agg1, W)
